# baseline (speedup 1.0000x reference)
import jax
import jax.numpy as jnp
from jax import lax
from jax.experimental import pallas as pl
from jax.experimental.pallas import tpu as pltpu
from jax.experimental.pallas import tpu_sc as plsc

N = 2048
F_IN = 128
H = 16
E = 65536
KF = N * (H - 2)

NC = 2
NS = 16
NW = NC * NS
CHUNK = 128
NCH = E // NW // CHUNK
NCHD = E // NS // CHUNK
NPT = N // NS

_sc_kernels_cache = []


def _rsqrt_sc(x):
    i = plsc.bitcast(x, jnp.int32)
    i = 0x5F3759DF - lax.shift_right_arithmetic(i, 1)
    y = plsc.bitcast(i, jnp.float32)
    for _ in range(3):
        y = y * (1.5 - 0.5 * x * y * y)
    return y


def _mega_body(h_hbm, ei_hbm,
               sp_hbm, dinv_hbm,
               dstall_v, of_v, zo_v, src_v, dst_v, rows_v, nh_v, nd_v, di_v,
               deg_sh, hd_sh, s_sh, sem, sem2):
    c = lax.axis_index("c")
    s = lax.axis_index("s")
    wid = s * NC + c

    pltpu.sync_copy(ei_hbm.at[1, pl.ds(s * NCHD, NCHD)], dstall_v)
    pltpu.sync_copy(ei_hbm.at[0, pl.ds(wid * NCH, NCH)], src_v)
    pltpu.sync_copy(ei_hbm.at[1, pl.ds(wid * NCH, NCH)], dst_v)
    pltpu.sync_copy(h_hbm.at[pl.ds(s * NPT, NPT)], nh_v)

    def _fill_flat(val):
        def body(k, _):
            of_v[pl.ds(k * H, H)] = jnp.full((H,), val, jnp.float32)
            return 0
        lax.fori_loop(0, CHUNK // H, body, 0)

    def _zero_rows():
        def body(k, _):
            zo_v[k] = jnp.zeros((H,), jnp.float32)
            return 0
        lax.fori_loop(0, NPT, body, 0)

    _fill_flat(0.0)
    _zero_rows()
    pltpu.sync_copy(of_v, deg_sh.at[pl.ds(s * NPT, NPT)])
    pltpu.sync_copy(zo_v, s_sh.at[pl.ds(s * NPT, NPT)])
    _fill_flat(1.0)

    plsc.subcore_barrier()

    deg_cps = [pltpu.make_async_copy(of_v, deg_sh.at[dstall_v.at[j]], sem2)
               for j in range(NCHD)]
    for cp in deg_cps:
        cp.start(add=True)
    for cp in deg_cps:
        cp.wait()
    plsc.subcore_barrier()

    pltpu.sync_copy(deg_sh.at[pl.ds(s * NPT, NPT)], nd_v)

    def _row(k, _):
        idx = jnp.full((H,), k, jnp.int32)
        deg_row = plsc.load_gather(nd_v, [idx]) + 1.0
        dinv_row = _rsqrt_sc(deg_row)
        di_v[k] = dinv_row
        nh_v[k] = nh_v[k] * dinv_row
        return 0

    lax.fori_loop(0, NPT, _row, 0)
    pltpu.sync_copy(nh_v, hd_sh.at[pl.ds(s * NPT, NPT)])

    @pl.when(c == 0)
    def _dinv_out():
        pltpu.sync_copy(di_v, dinv_hbm.at[pl.ds(s * NPT, NPT)])

    @pl.when(c == 0)
    def _seed_self_loop():
        pltpu.sync_copy(nh_v, s_sh.at[pl.ds(s * NPT, NPT)])

    plsc.subcore_barrier()

    copies = [pltpu.async_copy(hd_sh.at[src_v.at[j]], rows_v.at[j], sem)
              for j in range(NCH)]
    for cp in copies:
        cp.wait()
    sc_cps = [pltpu.make_async_copy(rows_v.at[j], s_sh.at[dst_v.at[j]], sem2)
              for j in range(NCH)]
    for cp in sc_cps:
        cp.start(add=True)
    for cp in sc_cps:
        cp.wait()
    plsc.subcore_barrier()

    @pl.when(s == 0)
    def _out():
        pltpu.sync_copy(s_sh, sp_hbm.at[c])


def _sc_kernels():
    if _sc_kernels_cache:
        return _sc_kernels_cache[0]
    mesh = plsc.VectorSubcoreMesh(core_axis_name="c", subcore_axis_name="s",
                                  num_cores=NC, num_subcores=NS)
    params = pltpu.CompilerParams(use_tc_tiling_on_sc=False,
                                  needs_layout_passes=False)
    mega = pl.kernel(
        _mega_body,
        out_type=(jax.ShapeDtypeStruct((NC, N, H), jnp.float32),
                  jax.ShapeDtypeStruct((N, H), jnp.float32)),
        mesh=mesh,
        compiler_params=params,
        scratch_types=[
            pltpu.VMEM((NCHD, CHUNK), jnp.int32),
            pltpu.VMEM((CHUNK,), jnp.float32),
            pltpu.VMEM((NPT, H), jnp.float32),
            pltpu.VMEM((NCH, CHUNK), jnp.int32),
            pltpu.VMEM((NCH, CHUNK), jnp.int32),
            pltpu.VMEM((NCH, CHUNK, H), jnp.float32),
            pltpu.VMEM((NPT, H), jnp.float32),
            pltpu.VMEM((NPT,), jnp.float32),
            pltpu.VMEM((NPT, H), jnp.float32),
            pltpu.VMEM_SHARED((N,), jnp.float32),
            pltpu.VMEM_SHARED((N, H), jnp.float32),
            pltpu.VMEM_SHARED((N, H), jnp.float32),
            pltpu.SemaphoreType.DMA,
            pltpu.SemaphoreType.DMA,
        ],
    )
    _sc_kernels_cache.append(mega)
    return mega


def _mm_body(x_ref, w_ref, h_ref):
    h_ref[...] = jnp.dot(x_ref[...], w_ref[...],
                         preferred_element_type=jnp.float32)


_mm = pl.pallas_call(
    _mm_body,
    out_shape=jax.ShapeDtypeStruct((N, H), jnp.float32),
)


def _finalize_body(sp_ref, dinv_ref, bg_ref, wt_ref, bt_ref, a_ref):
    s_tot = sp_ref[0] + sp_ref[1]
    g = jnp.maximum(dinv_ref[...] * s_tot + bg_ref[...], 0.0)
    conv = (wt_ref[0] * g[:, 0:H - 2] + wt_ref[1] * g[:, 1:H - 1]
            + wt_ref[2] * g[:, 2:H]) + bt_ref[0]
    a_ref[...] = jnp.maximum(conv, 0.0)


_finalize = pl.pallas_call(
    _finalize_body,
    in_specs=[
        pl.BlockSpec((NC, N, H), lambda: (0, 0, 0)),
        pl.BlockSpec((N, H), lambda: (0, 0)),
        pl.BlockSpec((1, H), lambda: (0, 0)),
        pl.BlockSpec(memory_space=pltpu.SMEM),
        pl.BlockSpec(memory_space=pltpu.SMEM),
    ],
    out_shape=jax.ShapeDtypeStruct((N, H - 2), jnp.float32),
)


GN = 32
NB = N // GN


def _fc_body(a_ref, w_ref, b_ref, o_ref):
    accs = [jnp.zeros((NB, 128), jnp.float32) for _ in range(4)]
    for t in range(KF // 128):
        sl = slice(t * 128, (t + 1) * 128)
        accs[t % 4] = accs[t % 4] + w_ref[:, sl] * a_ref[:, sl]
    acc = (accs[0] + accs[1]) + (accs[2] + accs[3])
    o_ref[...] = jnp.sum(acc, axis=1, keepdims=True) + b_ref[...]


_fc = pl.pallas_call(
    _fc_body,
    grid=(GN,),
    in_specs=[
        pl.BlockSpec((1, KF), lambda i: (0, 0)),
        pl.BlockSpec((NB, KF), lambda i: (i, 0)),
        pl.BlockSpec((NB, 1), lambda i: (i, 0)),
    ],
    out_specs=pl.BlockSpec((NB, 1), lambda i: (i, 0)),
    out_shape=jax.ShapeDtypeStruct((N, 1), jnp.float32),
    compiler_params=pltpu.CompilerParams(vmem_limit_bytes=100 * 1024 * 1024),
)


PROBE_ROWS = 8 * 88


PCOL = 7168
PCHUNKS = 12


def _probe_body(w_hbm, a_hbm, out_hbm, buf0, buf1, sema, semb):
    c = lax.axis_index("c")
    s = lax.axis_index("s")
    wid = s * NC + c
    rpt = 8 * (PCHUNKS // (KF // PCOL))
    base = wid * rpt

    def body(k, _):
        i0 = 2 * k
        i1 = 2 * k + 1
        cp0 = pltpu.async_copy(
            w_hbm.at[pl.ds(base + (i0 // 4) * 8, 8),
                     pl.ds((i0 % 4) * PCOL, PCOL)], buf0, sema)
        cp1 = pltpu.async_copy(
            w_hbm.at[pl.ds(base + (i1 // 4) * 8, 8),
                     pl.ds((i1 % 4) * PCOL, PCOL)], buf1, semb)
        cp0.wait()
        cp1.wait()
        return 0

    lax.fori_loop(0, PCHUNKS // 2, body, 0)

    @pl.when(wid == 0)
    def _done():
        pltpu.sync_copy(buf0.at[pl.ds(0, 8), pl.ds(0, 128)], out_hbm)


def _probe_kernel():
    mesh = plsc.VectorSubcoreMesh(core_axis_name="c", subcore_axis_name="s",
                                  num_cores=NC, num_subcores=NS)
    return pl.kernel(
        _probe_body,
        out_type=jax.ShapeDtypeStruct((8, 128), jnp.float32),
        mesh=mesh,
        compiler_params=pltpu.CompilerParams(use_tc_tiling_on_sc=True,
                                             has_side_effects=True),
        scratch_types=[
            pltpu.VMEM((8, PCOL), jnp.float32),
            pltpu.VMEM((8, PCOL), jnp.float32),
            pltpu.SemaphoreType.DMA,
            pltpu.SemaphoreType.DMA,
        ],
    )


def kernel(x, edge_index, W_gcn, b_gcn, w_tcn, b_tcn, W_fc, b_fc):
    ei3 = edge_index.reshape(2, E // CHUNK, CHUNK)

    h = _mm(x, W_gcn)
    sp, dinv = _sc_kernels()(h, ei3)
    a = _finalize(sp, dinv, b_gcn.reshape(1, H), w_tcn, b_tcn)
    flat = a.reshape(1, KF)
    _ = _probe_kernel()(W_fc, flat)
    out = _fc(flat, W_fc, b_fc.reshape(N, 1))
    return out.reshape(1, N)


# speedup vs baseline: 1.2585x; 1.2585x over previous
import jax
import jax.numpy as jnp
from jax import lax
from jax.experimental import pallas as pl
from jax.experimental.pallas import tpu as pltpu
from jax.experimental.pallas import tpu_sc as plsc

N = 2048
F_IN = 128
H = 16
E = 65536
KF = N * (H - 2)

NC = 2
NS = 16
NW = NC * NS
CHUNK = 128
NCH = E // NW // CHUNK
NCHD = E // NS // CHUNK
NPT = N // NS

_sc_kernels_cache = []


def _rsqrt_sc(x):
    i = plsc.bitcast(x, jnp.int32)
    i = 0x5F3759DF - lax.shift_right_arithmetic(i, 1)
    y = plsc.bitcast(i, jnp.float32)
    for _ in range(3):
        y = y * (1.5 - 0.5 * x * y * y)
    return y


def _mega_body(h_hbm, ei_hbm,
               sp_hbm,
               dstall_v, of_v, zo_v, src_v, dst_v, rows_v, nh_v, nd_v, di_v,
               deg_sh, hd_sh, s_sh, sem, sem2):
    c = lax.axis_index("c")
    s = lax.axis_index("s")
    wid = s * NC + c

    pltpu.sync_copy(ei_hbm.at[1, pl.ds(s * NCHD, NCHD)], dstall_v)
    pltpu.sync_copy(ei_hbm.at[0, pl.ds(wid * NCH, NCH)], src_v)
    pltpu.sync_copy(ei_hbm.at[1, pl.ds(wid * NCH, NCH)], dst_v)
    pltpu.sync_copy(h_hbm.at[pl.ds(s * NPT, NPT)], nh_v)

    def _fill_flat(val):
        def body(k, _):
            of_v[pl.ds(k * H, H)] = jnp.full((H,), val, jnp.float32)
            return 0
        lax.fori_loop(0, CHUNK // H, body, 0)

    def _zero_rows():
        def body(k, _):
            zo_v[k] = jnp.zeros((H,), jnp.float32)
            return 0
        lax.fori_loop(0, NPT, body, 0)

    _fill_flat(0.0)
    _zero_rows()
    pltpu.sync_copy(of_v, deg_sh.at[pl.ds(s * NPT, NPT)])
    pltpu.sync_copy(zo_v, s_sh.at[pl.ds(s * NPT, NPT)])
    _fill_flat(1.0)

    plsc.subcore_barrier()

    deg_cps = [pltpu.make_async_copy(of_v, deg_sh.at[dstall_v.at[j]], sem2)
               for j in range(NCHD)]
    for cp in deg_cps:
        cp.start(add=True)
    for cp in deg_cps:
        cp.wait()
    plsc.subcore_barrier()

    pltpu.sync_copy(deg_sh.at[pl.ds(s * NPT, NPT)], nd_v)

    def _row(k, _):
        idx = jnp.full((H,), k, jnp.int32)
        deg_row = plsc.load_gather(nd_v, [idx]) + 1.0
        dinv_row = _rsqrt_sc(deg_row)
        di_v[k] = dinv_row
        nh_v[k] = nh_v[k] * dinv_row
        return 0

    lax.fori_loop(0, NPT, _row, 0)
    pltpu.sync_copy(nh_v, hd_sh.at[pl.ds(s * NPT, NPT)])

    @pl.when(c == 0)
    def _seed_self_loop():
        pltpu.sync_copy(nh_v, s_sh.at[pl.ds(s * NPT, NPT)])

    plsc.subcore_barrier()

    copies = [pltpu.async_copy(hd_sh.at[src_v.at[j]], rows_v.at[j], sem)
              for j in range(NCH)]
    for cp in copies:
        cp.wait()
    sc_cps = [pltpu.make_async_copy(rows_v.at[j], s_sh.at[dst_v.at[j]], sem2)
              for j in range(NCH)]
    for cp in sc_cps:
        cp.start(add=True)
    for cp in sc_cps:
        cp.wait()
    plsc.subcore_barrier()

    pltpu.sync_copy(s_sh.at[pl.ds(s * NPT, NPT)], nh_v)

    def _scale(k, _):
        nh_v[k] = nh_v[k] * di_v[k]
        return 0

    lax.fori_loop(0, NPT, _scale, 0)
    pltpu.sync_copy(nh_v, sp_hbm.at[c, pl.ds(s * NPT, NPT)])


def _sc_kernels():
    if _sc_kernels_cache:
        return _sc_kernels_cache[0]
    mesh = plsc.VectorSubcoreMesh(core_axis_name="c", subcore_axis_name="s",
                                  num_cores=NC, num_subcores=NS)
    params = pltpu.CompilerParams(use_tc_tiling_on_sc=False,
                                  needs_layout_passes=False)
    mega = pl.kernel(
        _mega_body,
        out_type=jax.ShapeDtypeStruct((NC, N, H), jnp.float32),
        mesh=mesh,
        compiler_params=params,
        scratch_types=[
            pltpu.VMEM((NCHD, CHUNK), jnp.int32),
            pltpu.VMEM((CHUNK,), jnp.float32),
            pltpu.VMEM((NPT, H), jnp.float32),
            pltpu.VMEM((NCH, CHUNK), jnp.int32),
            pltpu.VMEM((NCH, CHUNK), jnp.int32),
            pltpu.VMEM((NCH, CHUNK, H), jnp.float32),
            pltpu.VMEM((NPT, H), jnp.float32),
            pltpu.VMEM((NPT,), jnp.float32),
            pltpu.VMEM((NPT, H), jnp.float32),
            pltpu.VMEM_SHARED((N,), jnp.float32),
            pltpu.VMEM_SHARED((N, H), jnp.float32),
            pltpu.VMEM_SHARED((N, H), jnp.float32),
            pltpu.SemaphoreType.DMA,
            pltpu.SemaphoreType.DMA,
        ],
    )
    _sc_kernels_cache.append(mega)
    return mega


def _mm_body(x_ref, w_ref, h_ref):
    h_ref[...] = jnp.dot(x_ref[...], w_ref[...],
                         preferred_element_type=jnp.float32)


_mm = pl.pallas_call(
    _mm_body,
    out_shape=jax.ShapeDtypeStruct((N, H), jnp.float32),
)


def _finalize_body(sp_ref, bg_ref, wt_ref, bt_ref, a_ref):
    g = jnp.maximum(sp_ref[0] + sp_ref[1] + bg_ref[...], 0.0)
    conv = (wt_ref[0] * g[:, 0:H - 2] + wt_ref[1] * g[:, 1:H - 1]
            + wt_ref[2] * g[:, 2:H]) + bt_ref[0]
    a_ref[...] = jnp.maximum(conv, 0.0)


_finalize = pl.pallas_call(
    _finalize_body,
    in_specs=[
        pl.BlockSpec((NC, N, H), lambda: (0, 0, 0)),
        pl.BlockSpec((1, H), lambda: (0, 0)),
        pl.BlockSpec(memory_space=pltpu.SMEM),
        pl.BlockSpec(memory_space=pltpu.SMEM),
    ],
    out_shape=jax.ShapeDtypeStruct((N, H - 2), jnp.float32),
)


GN = 32
NB = N // GN


def _fc_body(a_ref, w_ref, b_ref, o_ref):
    accs = [jnp.zeros((NB, 128), jnp.float32) for _ in range(4)]
    for t in range(KF // 128):
        sl = slice(t * 128, (t + 1) * 128)
        accs[t % 4] = accs[t % 4] + w_ref[:, sl] * a_ref[:, sl]
    acc = (accs[0] + accs[1]) + (accs[2] + accs[3])
    o_ref[...] = jnp.sum(acc, axis=1, keepdims=True) + b_ref[...]


_fc = pl.pallas_call(
    _fc_body,
    grid=(GN,),
    in_specs=[
        pl.BlockSpec((1, KF), lambda i: (0, 0)),
        pl.BlockSpec((NB, KF), lambda i: (i, 0)),
        pl.BlockSpec((NB, 1), lambda i: (i, 0)),
    ],
    out_specs=pl.BlockSpec((NB, 1), lambda i: (i, 0)),
    out_shape=jax.ShapeDtypeStruct((N, 1), jnp.float32),
    compiler_params=pltpu.CompilerParams(vmem_limit_bytes=100 * 1024 * 1024),
)


def kernel(x, edge_index, W_gcn, b_gcn, w_tcn, b_tcn, W_fc, b_fc):
    ei3 = edge_index.reshape(2, E // CHUNK, CHUNK)

    h = _mm(x, W_gcn)
    sp = _sc_kernels()(h, ei3)
    a = _finalize(sp, b_gcn.reshape(1, H), w_tcn, b_tcn)
    flat = a.reshape(1, KF)
    out = _fc(flat, W_fc, b_fc.reshape(N, 1))
    return out.reshape(1, N)


# speedup vs baseline: 1.2811x; 1.0180x over previous
import jax
import jax.numpy as jnp
from jax import lax
from jax.experimental import pallas as pl
from jax.experimental.pallas import tpu as pltpu
from jax.experimental.pallas import tpu_sc as plsc

N = 2048
F_IN = 128
H = 16
E = 65536
KF = N * (H - 2)

NC = 2
NS = 16
NW = NC * NS
CHUNK = 128
NCH = E // NW // CHUNK
NCHD = E // NS // CHUNK
NPT = N // NS

_sc_kernels_cache = []


def _rsqrt_sc(x):
    i = plsc.bitcast(x, jnp.int32)
    i = 0x5F3759DF - lax.shift_right_arithmetic(i, 1)
    y = plsc.bitcast(i, jnp.float32)
    for _ in range(3):
        y = y * (1.5 - 0.5 * x * y * y)
    return y


def _mega_body(h_hbm, ei_hbm, tcn_hbm,
               a_hbm,
               dstall_v, srcall_v, of_v, zo_v, rows_v, nh_v, nd_v, di_v,
               tcn_v, av_v, deg_sh, hd_sh, s_sh, sem, sem2):
    c = lax.axis_index("c")
    s = lax.axis_index("s")

    pltpu.sync_copy(ei_hbm.at[1, pl.ds(s * NCHD, NCHD)], dstall_v)
    pltpu.sync_copy(ei_hbm.at[0, pl.ds(s * NCHD, NCHD)], srcall_v)
    pltpu.sync_copy(h_hbm.at[pl.ds(s * NPT, NPT)], nh_v)
    pltpu.sync_copy(tcn_hbm, tcn_v)

    def _fill_flat(val):
        def body(k, _):
            of_v[pl.ds(k * H, H)] = jnp.full((H,), val, jnp.float32)
            return 0
        lax.fori_loop(0, CHUNK // H, body, 0)

    def _zero_rows():
        def body(k, _):
            zo_v[k] = jnp.zeros((H,), jnp.float32)
            return 0
        lax.fori_loop(0, NPT, body, 0)

    _fill_flat(0.0)
    _zero_rows()
    pltpu.sync_copy(of_v, deg_sh.at[pl.ds(s * NPT, NPT)])
    pltpu.sync_copy(zo_v, s_sh.at[pl.ds(s * NPT, NPT)])
    _fill_flat(1.0)

    plsc.subcore_barrier()

    deg_cps = [pltpu.make_async_copy(of_v, deg_sh.at[dstall_v.at[j]], sem2)
               for j in range(NCHD)]
    for cp in deg_cps:
        cp.start(add=True)
    for cp in deg_cps:
        cp.wait()
    plsc.subcore_barrier()

    pltpu.sync_copy(deg_sh.at[pl.ds(s * NPT, NPT)], nd_v)

    def _row(k, _):
        idx = jnp.full((H,), k, jnp.int32)
        deg_row = plsc.load_gather(nd_v, [idx]) + 1.0
        dinv_row = _rsqrt_sc(deg_row)
        di_v[k] = dinv_row
        nh_v[k] = nh_v[k] * dinv_row
        return 0

    lax.fori_loop(0, NPT, _row, 0)
    pltpu.sync_copy(nh_v, hd_sh.at[pl.ds(s * NPT, NPT)])
    pltpu.sync_copy(nh_v, s_sh.at[pl.ds(s * NPT, NPT)])
    plsc.subcore_barrier()

    copies = [pltpu.async_copy(hd_sh.at[srcall_v.at[j]], rows_v.at[j], sem)
              for j in range(NCHD)]
    for cp in copies:
        cp.wait()
    sc_cps = [pltpu.make_async_copy(rows_v.at[j], s_sh.at[dstall_v.at[j]],
                                    sem2)
              for j in range(NCHD)]
    for cp in sc_cps:
        cp.start(add=True)
    for cp in sc_cps:
        cp.wait()
    plsc.subcore_barrier()

    pltpu.sync_copy(s_sh.at[pl.ds(s * NPT, NPT)], nh_v)
    bg = tcn_v[4]

    def _g(k, _):
        nh_v[k] = jnp.maximum(nh_v[k] * di_v[k] + bg, 0.0)
        return 0

    lax.fori_loop(0, NPT, _g, 0)

    iota = lax.iota(jnp.int32, H)
    w0 = tcn_v[0]
    w1 = tcn_v[1]
    w2 = tcn_v[2]
    bt = tcn_v[3]
    for grp in range(NPT // H):
        rows = iota + grp * H
        cols = [plsc.load_gather(nh_v, [rows, jnp.full((H,), j, jnp.int32)])
                for j in range(H)]
        for j in range(H - 2):
            conv = w0 * cols[j] + w1 * cols[j + 1] + w2 * cols[j + 2] + bt
            aj = jnp.maximum(conv, 0.0)
            plsc.store_scatter(av_v, [rows * (H - 2) + j], aj)

    @pl.when(c == 0)
    def _a_out():
        pltpu.sync_copy(av_v, a_hbm.at[pl.ds(s * NPT * (H - 2), NPT * (H - 2))])


def _sc_kernels():
    if _sc_kernels_cache:
        return _sc_kernels_cache[0]
    mesh = plsc.VectorSubcoreMesh(core_axis_name="c", subcore_axis_name="s",
                                  num_cores=NC, num_subcores=NS)
    params = pltpu.CompilerParams(use_tc_tiling_on_sc=False,
                                  needs_layout_passes=False)
    mega = pl.kernel(
        _mega_body,
        out_type=jax.ShapeDtypeStruct((KF,), jnp.float32),
        mesh=mesh,
        compiler_params=params,
        scratch_types=[
            pltpu.VMEM((NCHD, CHUNK), jnp.int32),
            pltpu.VMEM((NCHD, CHUNK), jnp.int32),
            pltpu.VMEM((CHUNK,), jnp.float32),
            pltpu.VMEM((NPT, H), jnp.float32),
            pltpu.VMEM((NCHD, CHUNK, H), jnp.float32),
            pltpu.VMEM((NPT, H), jnp.float32),
            pltpu.VMEM((NPT,), jnp.float32),
            pltpu.VMEM((NPT, H), jnp.float32),
            pltpu.VMEM((5, H), jnp.float32),
            pltpu.VMEM((NPT * (H - 2),), jnp.float32),
            pltpu.VMEM_SHARED((N,), jnp.float32),
            pltpu.VMEM_SHARED((N, H), jnp.float32),
            pltpu.VMEM_SHARED((N, H), jnp.float32),
            pltpu.SemaphoreType.DMA,
            pltpu.SemaphoreType.DMA,
        ],
    )
    _sc_kernels_cache.append(mega)
    return mega


def _mm_body(x_ref, w_ref, h_ref):
    h_ref[...] = jnp.dot(x_ref[...], w_ref[...],
                         preferred_element_type=jnp.float32)


_mm = pl.pallas_call(
    _mm_body,
    out_shape=jax.ShapeDtypeStruct((N, H), jnp.float32),
)


GN = 32
NB = N // GN


def _fc_body(a_ref, w_ref, b_ref, o_ref):
    accs = [jnp.zeros((NB, 128), jnp.float32) for _ in range(4)]
    for t in range(KF // 128):
        sl = slice(t * 128, (t + 1) * 128)
        accs[t % 4] = accs[t % 4] + w_ref[:, sl] * a_ref[:, sl]
    acc = (accs[0] + accs[1]) + (accs[2] + accs[3])
    o_ref[...] = jnp.sum(acc, axis=1, keepdims=True) + b_ref[...]


_fc = pl.pallas_call(
    _fc_body,
    grid=(GN,),
    in_specs=[
        pl.BlockSpec((1, KF), lambda i: (0, 0)),
        pl.BlockSpec((NB, KF), lambda i: (i, 0)),
        pl.BlockSpec((NB, 1), lambda i: (i, 0)),
    ],
    out_specs=pl.BlockSpec((NB, 1), lambda i: (i, 0)),
    out_shape=jax.ShapeDtypeStruct((N, 1), jnp.float32),
    compiler_params=pltpu.CompilerParams(vmem_limit_bytes=100 * 1024 * 1024),
)


def kernel(x, edge_index, W_gcn, b_gcn, w_tcn, b_tcn, W_fc, b_fc):
    ei3 = edge_index.reshape(2, E // CHUNK, CHUNK)
    tcn = jnp.concatenate([
        jnp.broadcast_to(w_tcn[:, None], (3, H)),
        jnp.broadcast_to(b_tcn[:, None], (1, H)),
        b_gcn.reshape(1, H),
    ]).astype(jnp.float32)

    h = _mm(x, W_gcn)
    a_flat = _sc_kernels()(h, ei3, tcn)
    out = _fc(a_flat.reshape(1, KF), W_fc, b_fc.reshape(N, 1))
    return out.reshape(1, N)
